# baseline (speedup 1.0000x reference)
import functools

import jax
import jax.numpy as jnp
from jax import lax
from jax.experimental import pallas as pl
from jax.experimental.pallas import tpu as pltpu
from jax.experimental.pallas import tpu_sc as plsc

BATCH = 16384
D = 16
ROWS_PER_SLAB = 8
SLABS = 1000000 // ROWS_PER_SLAB
CHUNK = 128


def _constrain(W):
    c = jnp.linalg.norm(W, ord=2, axis=1, keepdims=True)
    c = jnp.where(c < 1.0, 1.0, c)
    return W / c


def _rsqrt_nr(x):
    ib = lax.bitcast_convert_type(x, jnp.int32)
    m = jnp.int32(0x5F3759DF) - lax.shift_right_arithmetic(ib, 1)
    y = lax.bitcast_convert_type(m, jnp.float32)
    y = y * (1.5 - 0.5 * x * y * y)
    y = y * (1.5 - 0.5 * x * y * y)
    return y


def _make_sc_kernel(n_workers, b_per_w):
    n_chunks = b_per_w // CHUNK
    groups_per_chunk = CHUNK // 16
    mesh = plsc.VectorSubcoreMesh(core_axis_name="c", subcore_axis_name="s")

    @functools.partial(
        pl.kernel,
        mesh=mesh,
        compiler_params=pltpu.CompilerParams(needs_layout_passes=False),
        out_type=(
            jax.ShapeDtypeStruct((BATCH,), jnp.float32),
            jax.ShapeDtypeStruct((n_workers * D,), jnp.float32),
            jax.ShapeDtypeStruct((n_workers * D,), jnp.float32),
        ),
        scratch_types=[
            pltpu.VMEM((14 * D,), jnp.float32),
            pltpu.VMEM((n_chunks, CHUNK), jnp.int32),
            pltpu.VMEM((n_chunks, CHUNK), jnp.int32),
            pltpu.VMEM((n_chunks, CHUNK), jnp.int32),
            pltpu.VMEM((n_chunks, CHUNK), jnp.int32),
            pltpu.VMEM((2, CHUNK, 128), jnp.float32),
            pltpu.VMEM((2, CHUNK, 128), jnp.float32),
            pltpu.VMEM((b_per_w,), jnp.float32),
            pltpu.VMEM((D,), jnp.float32),
            pltpu.VMEM((D,), jnp.float32),
            pltpu.SemaphoreType.DMA,
        ],
    )
    def k(users_hbm, items_hbm, u_tab, i_tab, params_hbm,
          out_hbm, pu_hbm, pi_hbm,
          params_v, idx_u, idx_i, sidx_u, sidx_i, slab_u, slab_i,
          out_v, accu_v, acci_v, sem):
        nc = lax.axis_index("c")
        ns = lax.axis_index("s")
        wid = ns * 2 + nc
        base = wid * b_per_w

        for j in range(n_chunks):
            pltpu.sync_copy(users_hbm.at[pl.ds(base + j * CHUNK, CHUNK)], idx_u.at[j])
            pltpu.sync_copy(items_hbm.at[pl.ds(base + j * CHUNK, CHUNK)], idx_i.at[j])
        pltpu.sync_copy(params_hbm, params_v)

        for j in range(n_chunks):
            for s in range(CHUNK // 16):
                sl = pl.ds(s * 16, 16)
                sidx_u[j, sl] = lax.shift_right_logical(idx_u[j, sl], ROWS_PER_SLAB.bit_length() - 1)
                sidx_i[j, sl] = lax.shift_right_logical(idx_i[j, sl], ROWS_PER_SLAB.bit_length() - 1)

        def fire(j):
            return (pltpu.async_copy(u_tab.at[sidx_u.at[j]], slab_u.at[j % 2], sem),
                    pltpu.async_copy(i_tab.at[sidx_i.at[j]], slab_i.at[j % 2], sem))

        p = [params_v[pl.ds(r * D, D)] for r in range(14)]
        wus = [p[0][c] for c in range(D)]
        wis = [p[1][c] for c in range(D)]
        wms = [p[2][c] for c in range(D)]
        wds = [p[3][c] for c in range(D)]
        u0, u1, u2, cp, sp = p[4], p[5], p[6], p[7], p[8]
        q0, q1, q2, cq, sq = p[9], p[10], p[11], p[12], p[13]
        lane = lax.iota(jnp.int32, 16)

        def trans(e, t0, t1, t2, ca, sa):
            ab = jnp.abs(e)
            x = ab + 1e-7
            s = x * _rsqrt_nr(x)
            sqr = e * e
            unary = t0 * s + t1 * ab + t2 * sqr
            assist = ca + sa * jnp.sign(e)
            return unary * assist, sqr

        au = jnp.zeros((16,), jnp.float32)
        ai = jnp.zeros((16,), jnp.float32)

        inflight = fire(0)
        for j in range(n_chunks):
            nxt = fire(j + 1) if j + 1 < n_chunks else None
            inflight[0].wait()
            inflight[1].wait()
            bu = slab_u.at[j % 2]
            bi = slab_i.at[j % 2]

            def group_body(g, carry, j=j, bu=bu, bi=bi):
                au, ai = carry
                sl = pl.ds(g * 16, 16)
                offu = lax.shift_left(jnp.bitwise_and(idx_u[j, sl], ROWS_PER_SLAB - 1), 4)
                offi = lax.shift_left(jnp.bitwise_and(idx_i[j, sl], ROWS_PER_SLAB - 1), 4)
                res = jnp.zeros((16,), jnp.float32)
                for c in range(D):
                    cu = plsc.load_gather(bu, [lane + g * 16, offu + c])
                    ci = plsc.load_gather(bi, [lane + g * 16, offi + c])
                    tu, squ = trans(cu, u0, u1, u2, cp, sp)
                    ti, sqi = trans(ci, q0, q1, q2, cq, sq)
                    au = au + squ
                    ai = ai + sqi
                    res = (res + tu * wus[c] + ti * wis[c]
                           + (tu * ti) * wms[c] + jnp.abs(tu - ti) * wds[c])
                out_v[pl.ds(j * CHUNK + g * 16, 16)] = res
                return au, ai

            au, ai = lax.fori_loop(0, groups_per_chunk, group_body, (au, ai))
            inflight = nxt

        accu_v[...] = au
        acci_v[...] = ai

        pltpu.sync_copy(out_v, out_hbm.at[pl.ds(base, b_per_w)])
        pltpu.sync_copy(accu_v, pu_hbm.at[pl.ds(wid * D, D)])
        pltpu.sync_copy(acci_v, pi_hbm.at[pl.ds(wid * D, D)])

    return k


def kernel(users, items, U, I, a_unary_p, a_unary_q, a_assist_p, a_assist_q,
           a_binary, W0, W1, W2, W3, W4):
    W0, W1, W2, W3, W4 = map(_constrain, (W0, W1, W2, W3, W4))
    a = a_binary
    half = 0.5 * (a[2] * W2[0] + a[3] * W3[0])
    wu = a[0] * W0[0] + half + a[4] * W4[0, :D]
    wi = a[0] * W0[0] + half + a[4] * W4[0, D:]
    wm = a[1] * W1[0]
    wd = 0.5 * (a[2] * W2[0] - a[3] * W3[0])
    sp = jax.nn.softmax(a_assist_p)
    sq = jax.nn.softmax(a_assist_q)

    def splat(s):
        return jnp.full((D,), s, jnp.float32)

    params = jnp.concatenate([
        wu, wi, wm, wd,
        splat(a_unary_p[0]), splat(a_unary_p[1]), splat(a_unary_p[2]),
        splat(sp[0] - sp[1]), splat(sp[2]),
        splat(a_unary_q[0]), splat(a_unary_q[1]), splat(a_unary_q[2]),
        splat(sq[0] - sq[1]), splat(sq[2]),
    ])

    info = plsc.get_sparse_core_info()
    n_workers = info.num_cores * info.num_subcores
    b_per_w = BATCH // n_workers

    k = _make_sc_kernel(n_workers, b_per_w)
    out, pu, pi = k(users.astype(jnp.int32), items.astype(jnp.int32),
                    U.reshape(SLABS, ROWS_PER_SLAB * D),
                    I.reshape(SLABS, ROWS_PER_SLAB * D),
                    params)

    inferences = out.reshape(BATCH, 1)
    regs = 0.01 * (jnp.sqrt(jnp.sum(pu)) + jnp.sqrt(jnp.sum(pi)))
    return inferences, regs
